# Initial kernel scaffold; baseline (speedup 1.0000x reference)
import jax
import jax.numpy as jnp
from jax.experimental import pallas as pl


def kernel(x, m, root, edge_index, W, b, depth):
    raise NotImplementedError("write your pallas kernel here")


# speedup vs baseline: 4.5655x; 4.5655x over previous
import functools

import jax
import jax.numpy as jnp
from jax import lax
from jax.experimental import pallas as pl
from jax.experimental.pallas import tpu as pltpu
from jax.experimental.pallas import tpu_sc as plsc

NC = 2
NS = 16
NW = NC * NS
CH = 80
RING = 8
NSTRIPE = 8


def _sc_edge_kernel(e, npad, root, m, x, msg, src2d, dst2d, zacc, onesb):
    n, d = root.shape
    _, nchunk, ch = src2d.shape
    ngrp = nchunk // RING
    zr = npad // NS
    assert zr // ch == NSTRIPE

    mesh = plsc.VectorSubcoreMesh(
        core_axis_name="c", subcore_axis_name="s", num_cores=NC, num_subcores=NS
    )

    @functools.partial(
        pl.kernel,
        out_type=(
            jax.ShapeDtypeStruct((e, 3 * d), jnp.float32),
            jax.ShapeDtypeStruct((NC * npad, d), jnp.float32),
            jax.ShapeDtypeStruct((NC * npad, d), jnp.float32),
            jax.ShapeDtypeStruct((NC * npad, d), jnp.float32),
        ),
        mesh=mesh,
        scratch_types=(
            pltpu.VMEM_SHARED((npad, d), jnp.float32),
            pltpu.VMEM((RING, ch), jnp.int32),
            pltpu.VMEM((RING, ch), jnp.int32),
            pltpu.VMEM((ch, d), jnp.float32),
            pltpu.VMEM((ch, d), jnp.float32),
            pltpu.VMEM((ch, d), jnp.float32),
            pltpu.VMEM((ch, d), jnp.float32),
            pltpu.SemaphoreType.DMA,
            pltpu.SemaphoreType.DMA,
            pltpu.SemaphoreType.DMA,
        ),
    )
    def k(root_h, m_h, x_h, msg_h, src_h, dst_h, zacc_h, ones_h,
          out_h, pr_h, pm_h, pd_h,
          acc, srcb, dstb, rowa, rowb, rowc, onesr,
          sema, semb, semc):
        cid = lax.axis_index("c")
        sid = lax.axis_index("s")
        wid = cid * NS + sid
        ebase = wid * nchunk * ch

        def zero_acc_stripe():
            pltpu.sync_copy(zacc_h, rowa)

            def zs(b, carry):
                pltpu.sync_copy(rowa, acc.at[pl.ds(sid * zr + b * ch, ch)])
                return carry

            lax.fori_loop(0, NSTRIPE, zs, 0)

        def flush_acc_stripe(dst_h_ref, base):
            def fl(b, carry):
                pltpu.sync_copy(acc.at[pl.ds(sid * zr + b * ch, ch)], rowa)
                pltpu.sync_copy(rowa, dst_h_ref.at[pl.ds(base + b * ch, ch)])
                return carry

            lax.fori_loop(0, NSTRIPE, fl, 0)

        zero_acc_stripe()
        pltpu.sync_copy(ones_h, onesr)
        plsc.subcore_barrier()

        def phase1_grp(g, carry):
            @pl.when(ebase + g * RING * ch < e)
            def _():
                pltpu.sync_copy(src_h.at[wid, pl.ds(g * RING, RING)], srcb)
                pltpu.sync_copy(dst_h.at[wid, pl.ds(g * RING, RING)], dstb)

                def phase1(k, carry2):
                    j = g * RING + k
                    sidx = srcb.at[k]
                    didx = dstb.at[k]
                    row = pl.ds(ebase + j * ch, ch)
                    cp_a = pltpu.async_copy(root_h.at[sidx], rowa, sema)
                    cp_b = pltpu.async_copy(m_h.at[sidx], rowb, semb)
                    cp_c = pltpu.async_copy(x_h.at[didx], rowc, semc)
                    cp_a.wait()
                    wa = pltpu.async_copy(rowa, out_h.at[row, pl.ds(0, d)],
                                          sema)
                    pltpu.sync_copy(rowa, acc.at[didx], add=True)
                    cp_b.wait()
                    wb = pltpu.async_copy(rowb, out_h.at[row, pl.ds(d, d)],
                                          semb)
                    cp_c.wait()
                    wc = pltpu.async_copy(rowc, out_h.at[row, pl.ds(2 * d, d)],
                                          semc)
                    wa.wait()
                    wb.wait()
                    wc.wait()
                    return carry2

                lax.fori_loop(0, RING, phase1, 0)

            return carry

        lax.fori_loop(0, ngrp, phase1_grp, 0)
        plsc.subcore_barrier()

        flush_acc_stripe(pr_h, cid * npad + sid * zr)
        zero_acc_stripe()
        plsc.subcore_barrier()

        def phase2_grp(g, carry):
            @pl.when(ebase + g * RING * ch < e)
            def _():
                pltpu.sync_copy(src_h.at[wid, pl.ds(g * RING, RING)], srcb)
                pltpu.sync_copy(dst_h.at[wid, pl.ds(g * RING, RING)], dstb)

                def phase2(k, carry2):
                    pltpu.async_copy(msg_h.at[srcb.at[k]], rowa, sema).wait()
                    pltpu.sync_copy(rowa, acc.at[dstb.at[k]], add=True)
                    return carry2

                lax.fori_loop(0, RING, phase2, 0)

            return carry

        lax.fori_loop(0, ngrp, phase2_grp, 0)
        plsc.subcore_barrier()
        flush_acc_stripe(pm_h, cid * npad + sid * zr)
        zero_acc_stripe()
        plsc.subcore_barrier()

        def phase3_grp(g, carry):
            @pl.when(ebase + g * RING * ch < e)
            def _():
                pltpu.sync_copy(dst_h.at[wid, pl.ds(g * RING, RING)], dstb)

                def phase3(k, carry2):
                    pltpu.sync_copy(onesr, acc.at[dstb.at[k]], add=True)
                    return carry2

                lax.fori_loop(0, RING, phase3, 0)

            return carry

        lax.fori_loop(0, ngrp, phase3_grp, 0)
        plsc.subcore_barrier()
        flush_acc_stripe(pd_h, cid * npad + sid * zr)

    return k(root, m, x, msg, src2d, dst2d, zacc, onesb)


def _tc_combine(x, m, root, w, b2, pr0, pr1, pm0, pm1, pd0, pd1):
    n, d = x.shape
    bn = 1000
    grid = (n // bn,)

    def body(x_r, m_r, root_r, w_r, b_r, pr0_r, pr1_r, pm0_r, pm1_r,
             pd0_r, pd1_r, newm_r, newroot_r):
        degv = pd0_r[...] + pd1_r[...]
        deg = degv[:, 0:1]
        denom = jnp.maximum(deg, 1.0)
        has = deg > 0.0
        segm = pm0_r[...] + pm1_r[...]
        segr = pr0_r[...] + pr1_r[...]
        xm = x_r[...] + segm / denom
        h = lax.dot_general(xm, w_r[...], (((1,), (1,)), ((), ())),
                            preferred_element_type=jnp.float32)
        h = h + 2.0 * b_r[...]
        newm_r[...] = jnp.where(has, jnp.maximum(h, 0.0), m_r[...])
        newroot_r[...] = jnp.where(has, segr / denom, root_r[...])

    row_spec = pl.BlockSpec((bn, d), lambda i: (i, 0))
    deg_spec = pl.BlockSpec((bn, d), lambda i: (i, 0))
    full_spec = pl.BlockSpec((d, d), lambda i: (0, 0))
    b_spec = pl.BlockSpec((1, d), lambda i: (0, 0))
    return pl.pallas_call(
        body,
        grid=grid,
        in_specs=[row_spec, row_spec, row_spec, full_spec, b_spec,
                  row_spec, row_spec, row_spec, row_spec, deg_spec, deg_spec],
        out_specs=[row_spec, row_spec],
        out_shape=[
            jax.ShapeDtypeStruct((n, d), jnp.float32),
            jax.ShapeDtypeStruct((n, d), jnp.float32),
        ],
    )(x, m, root, w, b2, pr0, pr1, pm0, pm1, pd0, pd1)


def kernel(x, m, root, edge_index, W, b, depth):
    n, d = x.shape
    e = edge_index.shape[1]
    assert e % CH == 0 and n % NS == 0

    ept = -(-e // (NW * CH * RING)) * (CH * RING)
    epad = NW * ept
    src2d = jnp.pad(edge_index[0], (0, epad - e)).reshape(NW, ept // CH, CH)
    dst2d = jnp.pad(edge_index[1], (0, epad - e)).reshape(NW, ept // CH, CH)
    msg = jnp.where(depth == 1, x, m)
    npad = NS * CH * NSTRIPE
    assert npad >= n
    zacc = jnp.zeros((CH, d), jnp.float32)
    onesb = jnp.ones((CH, d), jnp.float32)

    out, pr, pm, pd = _sc_edge_kernel(e, npad, root, m, x, msg, src2d, dst2d,
                                      zacc, onesb)
    new_m, new_root = _tc_combine(
        x, m, root, W, b.reshape(1, d),
        pr[:n], pr[npad:npad + n], pm[:n], pm[npad:npad + n],
        pd[:n], pd[npad:npad + n])
    return out, new_m, new_root


# speedup vs baseline: 5.2193x; 1.1432x over previous
import functools

import jax
import jax.numpy as jnp
from jax import lax
from jax.experimental import pallas as pl
from jax.experimental.pallas import tpu as pltpu
from jax.experimental.pallas import tpu_sc as plsc

NC = 2
NS = 16
NW = NC * NS
CH = 80
RING = 16
NSTRIPE = 8


def _sc_edge_kernel(e, npad, root, m, x, msg, src2d, dst2d, zacc, onesb):
    n, d = root.shape
    _, nchunk, ch = src2d.shape
    ngrp = nchunk // RING
    zr = npad // NS
    assert zr // ch == NSTRIPE

    mesh = plsc.VectorSubcoreMesh(
        core_axis_name="c", subcore_axis_name="s", num_cores=NC, num_subcores=NS
    )

    @functools.partial(
        pl.kernel,
        out_type=(
            jax.ShapeDtypeStruct((e, 3 * d), jnp.float32),
            jax.ShapeDtypeStruct((NC * npad, d), jnp.float32),
            jax.ShapeDtypeStruct((NC * npad, d), jnp.float32),
            jax.ShapeDtypeStruct((NC * npad, d), jnp.float32),
        ),
        mesh=mesh,
        scratch_types=(
            pltpu.VMEM_SHARED((npad, d), jnp.float32),
            pltpu.VMEM((RING, ch), jnp.int32),
            pltpu.VMEM((RING, ch), jnp.int32),
            pltpu.VMEM((ch, d), jnp.float32),
            pltpu.VMEM((ch, d), jnp.float32),
            pltpu.VMEM((ch, d), jnp.float32),
            pltpu.VMEM((ch, d), jnp.float32),
            pltpu.SemaphoreType.DMA,
            pltpu.SemaphoreType.DMA,
            pltpu.SemaphoreType.DMA,
        ),
    )
    def k(root_h, m_h, x_h, msg_h, src_h, dst_h, zacc_h, ones_h,
          out_h, pr_h, pm_h, pd_h,
          acc, srcb, dstb, rowa, rowb, rowc, onesr,
          sema, semb, semc):
        cid = lax.axis_index("c")
        sid = lax.axis_index("s")
        wid = cid * NS + sid
        ebase = wid * nchunk * ch

        def zero_acc_stripe():
            pltpu.sync_copy(zacc_h, rowa)

            def zs(b, carry):
                pltpu.sync_copy(rowa, acc.at[pl.ds(sid * zr + b * ch, ch)])
                return carry

            lax.fori_loop(0, NSTRIPE, zs, 0)

        def flush_acc_stripe(dst_h_ref, base):
            def fl(b, carry):
                pltpu.sync_copy(acc.at[pl.ds(sid * zr + b * ch, ch)], rowa)
                pltpu.sync_copy(rowa, dst_h_ref.at[pl.ds(base + b * ch, ch)])
                return carry

            lax.fori_loop(0, NSTRIPE, fl, 0)

        zero_acc_stripe()
        pltpu.sync_copy(ones_h, onesr)
        plsc.subcore_barrier()

        def phase1_grp(g, carry):
            @pl.when(ebase + g * RING * ch < e)
            def _():
                pltpu.sync_copy(src_h.at[wid, pl.ds(g * RING, RING)], srcb)
                pltpu.sync_copy(dst_h.at[wid, pl.ds(g * RING, RING)], dstb)

                def phase1(k, carry2):
                    j = g * RING + k
                    sidx = srcb.at[k]
                    didx = dstb.at[k]
                    row = pl.ds(ebase + j * ch, ch)
                    cp_a = pltpu.async_copy(root_h.at[sidx], rowa, sema)
                    cp_b = pltpu.async_copy(m_h.at[sidx], rowb, semb)
                    cp_c = pltpu.async_copy(x_h.at[didx], rowc, semc)
                    cp_a.wait()
                    wa = pltpu.async_copy(rowa, out_h.at[row, pl.ds(0, d)],
                                          sema)
                    pltpu.sync_copy(rowa, acc.at[didx], add=True)
                    cp_b.wait()
                    wb = pltpu.async_copy(rowb, out_h.at[row, pl.ds(d, d)],
                                          semb)
                    cp_c.wait()
                    wc = pltpu.async_copy(rowc, out_h.at[row, pl.ds(2 * d, d)],
                                          semc)
                    wa.wait()
                    wb.wait()
                    wc.wait()
                    return carry2

                lax.fori_loop(0, RING, phase1, 0)

            return carry

        lax.fori_loop(0, ngrp, phase1_grp, 0)
        plsc.subcore_barrier()

        flush_acc_stripe(pr_h, cid * npad + sid * zr)
        zero_acc_stripe()
        plsc.subcore_barrier()

        def phase2_grp(g, carry):
            @pl.when(ebase + g * RING * ch < e)
            def _():
                pltpu.sync_copy(src_h.at[wid, pl.ds(g * RING, RING)], srcb)
                pltpu.sync_copy(dst_h.at[wid, pl.ds(g * RING, RING)], dstb)
                bufs2 = (rowa, rowb)
                sems2 = (sema, semb)
                gd = [None] * RING
                gd[0] = pltpu.async_copy(msg_h.at[srcb.at[0]], rowa, sema)
                for k in range(RING):
                    if k + 1 < RING:
                        gd[k + 1] = pltpu.async_copy(
                            msg_h.at[srcb.at[k + 1]], bufs2[(k + 1) % 2],
                            sems2[(k + 1) % 2])
                    gd[k].wait()
                    pltpu.sync_copy(bufs2[k % 2], acc.at[dstb.at[k]], add=True)

            return carry

        lax.fori_loop(0, ngrp, phase2_grp, 0)
        plsc.subcore_barrier()
        flush_acc_stripe(pm_h, cid * npad + sid * zr)
        zero_acc_stripe()
        plsc.subcore_barrier()

        def phase3_grp(g, carry):
            @pl.when(ebase + g * RING * ch < e)
            def _():
                pltpu.sync_copy(dst_h.at[wid, pl.ds(g * RING, RING)], dstb)

                def phase3(k, carry2):
                    pltpu.sync_copy(onesr, acc.at[dstb.at[k]], add=True)
                    return carry2

                lax.fori_loop(0, RING, phase3, 0)

            return carry

        lax.fori_loop(0, ngrp, phase3_grp, 0)
        plsc.subcore_barrier()
        flush_acc_stripe(pd_h, cid * npad + sid * zr)

    return k(root, m, x, msg, src2d, dst2d, zacc, onesb)


def _tc_combine(x, m, root, w, b2, pr0, pr1, pm0, pm1, pd0, pd1):
    n, d = x.shape
    bn = 1000
    grid = (n // bn,)

    def body(x_r, m_r, root_r, w_r, b_r, pr0_r, pr1_r, pm0_r, pm1_r,
             pd0_r, pd1_r, newm_r, newroot_r):
        degv = pd0_r[...] + pd1_r[...]
        deg = degv[:, 0:1]
        denom = jnp.maximum(deg, 1.0)
        has = deg > 0.0
        segm = pm0_r[...] + pm1_r[...]
        segr = pr0_r[...] + pr1_r[...]
        xm = x_r[...] + segm / denom
        h = lax.dot_general(xm, w_r[...], (((1,), (1,)), ((), ())),
                            preferred_element_type=jnp.float32)
        h = h + 2.0 * b_r[...]
        newm_r[...] = jnp.where(has, jnp.maximum(h, 0.0), m_r[...])
        newroot_r[...] = jnp.where(has, segr / denom, root_r[...])

    row_spec = pl.BlockSpec((bn, d), lambda i: (i, 0))
    deg_spec = pl.BlockSpec((bn, d), lambda i: (i, 0))
    full_spec = pl.BlockSpec((d, d), lambda i: (0, 0))
    b_spec = pl.BlockSpec((1, d), lambda i: (0, 0))
    return pl.pallas_call(
        body,
        grid=grid,
        in_specs=[row_spec, row_spec, row_spec, full_spec, b_spec,
                  row_spec, row_spec, row_spec, row_spec, deg_spec, deg_spec],
        out_specs=[row_spec, row_spec],
        out_shape=[
            jax.ShapeDtypeStruct((n, d), jnp.float32),
            jax.ShapeDtypeStruct((n, d), jnp.float32),
        ],
    )(x, m, root, w, b2, pr0, pr1, pm0, pm1, pd0, pd1)


def kernel(x, m, root, edge_index, W, b, depth):
    n, d = x.shape
    e = edge_index.shape[1]
    assert e % CH == 0 and n % NS == 0

    ept = -(-e // (NW * CH * RING)) * (CH * RING)
    epad = NW * ept
    src2d = jnp.pad(edge_index[0], (0, epad - e)).reshape(NW, ept // CH, CH)
    dst2d = jnp.pad(edge_index[1], (0, epad - e)).reshape(NW, ept // CH, CH)
    msg = jnp.where(depth == 1, x, m)
    npad = NS * CH * NSTRIPE
    assert npad >= n
    zacc = jnp.zeros((CH, d), jnp.float32)
    onesb = jnp.ones((CH, d), jnp.float32)

    out, pr, pm, pd = _sc_edge_kernel(e, npad, root, m, x, msg, src2d, dst2d,
                                      zacc, onesb)
    new_m, new_root = _tc_combine(
        x, m, root, W, b.reshape(1, d),
        pr[:n], pr[npad:npad + n], pm[:n], pm[npad:npad + n],
        pd[:n], pd[npad:npad + n])
    return out, new_m, new_root


# speedup vs baseline: 5.5390x; 1.0613x over previous
import functools

import jax
import jax.numpy as jnp
from jax import lax
from jax.experimental import pallas as pl
from jax.experimental.pallas import tpu as pltpu
from jax.experimental.pallas import tpu_sc as plsc

NC = 2
NS = 16
NW = NC * NS
CH = 128
RING = 8
NSTRIPE = 5


def _sc_edge_kernel(e, npad, root, m, x, msg, src2d, dst2d, zacc, onesb):
    n, d = root.shape
    _, nchunk, ch = src2d.shape
    ngrp = nchunk // RING
    zr = npad // NS
    assert zr // ch == NSTRIPE

    mesh = plsc.VectorSubcoreMesh(
        core_axis_name="c", subcore_axis_name="s", num_cores=NC, num_subcores=NS
    )

    @functools.partial(
        pl.kernel,
        out_type=(
            jax.ShapeDtypeStruct((e, 3 * d), jnp.float32),
            jax.ShapeDtypeStruct((NC * npad, d), jnp.float32),
            jax.ShapeDtypeStruct((NC * npad, d), jnp.float32),
            jax.ShapeDtypeStruct((NC * npad, d), jnp.float32),
        ),
        mesh=mesh,
        scratch_types=(
            pltpu.VMEM_SHARED((npad, d), jnp.float32),
            pltpu.VMEM((RING, ch), jnp.int32),
            pltpu.VMEM((RING, ch), jnp.int32),
            pltpu.VMEM((ch, d), jnp.float32),
            pltpu.VMEM((ch, d), jnp.float32),
            pltpu.SemaphoreType.DMA,
            pltpu.SemaphoreType.DMA,
            pltpu.SemaphoreType.DMA,
            pltpu.SemaphoreType.DMA,
        ),
    )
    def k(root_h, m_h, x_h, msg_h, src_h, dst_h, zacc_h, ones_h,
          out_h, pr_h, pm_h, pd_h,
          acc, irb, dstb, s0, s1, sg0, sg1, sw0, sw1):
        cid = lax.axis_index("c")
        sid = lax.axis_index("s")
        wid = cid * NS + sid
        ebase = wid * nchunk * ch
        slots = (s0, s1)
        gsem = (sg0, sg1)
        wsem = (sw0, sw1)

        def zero_acc_stripe():
            pltpu.sync_copy(zacc_h, s0)

            def zs(b, carry):
                pltpu.sync_copy(s0, acc.at[pl.ds(sid * zr + b * ch, ch)])
                return carry

            lax.fori_loop(0, NSTRIPE, zs, 0)

        def flush_acc_stripe(dst_h_ref, base):
            def fl(b, carry):
                pltpu.sync_copy(acc.at[pl.ds(sid * zr + b * ch, ch)], s0)
                pltpu.sync_copy(s0, dst_h_ref.at[pl.ds(base + b * ch, ch)])
                return carry

            lax.fori_loop(0, NSTRIPE, fl, 0)

        def edge_phase(tab_h, idx_is_dst, colbase, do_add):
            ih = dst_h if idx_is_dst else src_h

            def chunk_simple(k_, jb):
                sl = 0
                pltpu.async_copy(tab_h.at[irb.at[k_]], slots[sl],
                                 gsem[sl]).wait()
                if colbase is not None:
                    row = pl.ds(jb + k_ * ch, ch)
                    pltpu.sync_copy(slots[sl],
                                    out_h.at[row, pl.ds(colbase * d, d)])
                if do_add:
                    didx = dstb.at[k_] if not idx_is_dst else irb.at[k_]
                    pltpu.sync_copy(slots[sl], acc.at[didx], add=True)

            def grp(g, carry):
                nreal = jnp.clip((e - ebase) // ch - g * RING, 0, RING)

                @pl.when(nreal > 0)
                def _():
                    pltpu.sync_copy(ih.at[wid, pl.ds(g * RING, RING)], irb)
                    if do_add and not idx_is_dst:
                        pltpu.sync_copy(
                            dst_h.at[wid, pl.ds(g * RING, RING)], dstb)
                    jb = ebase + g * RING * ch

                    @pl.when(nreal == RING)
                    def _():
                        gd = [None] * RING
                        wd = [None] * RING
                        gd[0] = pltpu.async_copy(
                            tab_h.at[irb.at[0]], slots[0], gsem[0])
                        for k_ in range(RING):
                            sl = k_ % 2
                            if k_ + 1 < RING:
                                if k_ >= 1 and wd[k_ - 1] is not None:
                                    wd[k_ - 1].wait()
                                gd[k_ + 1] = pltpu.async_copy(
                                    tab_h.at[irb.at[k_ + 1]],
                                    slots[(k_ + 1) % 2], gsem[(k_ + 1) % 2])
                            gd[k_].wait()
                            if colbase is not None:
                                row = pl.ds(jb + k_ * ch, ch)
                                wd[k_] = pltpu.async_copy(
                                    slots[sl],
                                    out_h.at[row, pl.ds(colbase * d, d)],
                                    wsem[sl])
                            if do_add:
                                didx = (dstb.at[k_] if not idx_is_dst
                                        else irb.at[k_])
                                pltpu.sync_copy(slots[sl], acc.at[didx],
                                                add=True)
                        for k_ in (RING - 2, RING - 1):
                            if wd[k_] is not None:
                                wd[k_].wait()

                    @pl.when(nreal < RING)
                    def _():
                        def simple(k_, c2):
                            chunk_simple(k_, jb)
                            return c2

                        lax.fori_loop(0, nreal, simple, 0)

                return carry

            lax.fori_loop(0, ngrp, grp, 0)

        zero_acc_stripe()
        plsc.subcore_barrier()

        edge_phase(root_h, False, 0, True)
        plsc.subcore_barrier()
        flush_acc_stripe(pr_h, cid * npad + sid * zr)
        zero_acc_stripe()
        plsc.subcore_barrier()

        edge_phase(m_h, False, 1, False)
        edge_phase(x_h, True, 2, False)
        edge_phase(msg_h, False, None, True)
        plsc.subcore_barrier()
        flush_acc_stripe(pm_h, cid * npad + sid * zr)
        zero_acc_stripe()
        plsc.subcore_barrier()

        pltpu.sync_copy(ones_h, s0)

        def deg_grp(g, carry):
            nreal = jnp.clip((e - ebase) // ch - g * RING, 0, RING)

            @pl.when(nreal > 0)
            def _():
                pltpu.sync_copy(dst_h.at[wid, pl.ds(g * RING, RING)], dstb)

                def f(k_, c2):
                    pltpu.sync_copy(s0, acc.at[dstb.at[k_]], add=True)
                    return c2

                lax.fori_loop(0, nreal, f, 0)

            return carry

        lax.fori_loop(0, ngrp, deg_grp, 0)
        plsc.subcore_barrier()
        flush_acc_stripe(pd_h, cid * npad + sid * zr)

    return k(root, m, x, msg, src2d, dst2d, zacc, onesb)


def _tc_combine(x, m, root, w, b2, pr0, pr1, pm0, pm1, pd0, pd1):
    n, d = x.shape
    bn = 1000
    grid = (n // bn,)

    def body(x_r, m_r, root_r, w_r, b_r, pr0_r, pr1_r, pm0_r, pm1_r,
             pd0_r, pd1_r, newm_r, newroot_r):
        degv = pd0_r[...] + pd1_r[...]
        deg = degv[:, 0:1]
        denom = jnp.maximum(deg, 1.0)
        has = deg > 0.0
        segm = pm0_r[...] + pm1_r[...]
        segr = pr0_r[...] + pr1_r[...]
        xm = x_r[...] + segm / denom
        h = lax.dot_general(xm, w_r[...], (((1,), (1,)), ((), ())),
                            preferred_element_type=jnp.float32)
        h = h + 2.0 * b_r[...]
        newm_r[...] = jnp.where(has, jnp.maximum(h, 0.0), m_r[...])
        newroot_r[...] = jnp.where(has, segr / denom, root_r[...])

    row_spec = pl.BlockSpec((bn, d), lambda i: (i, 0))
    deg_spec = pl.BlockSpec((bn, d), lambda i: (i, 0))
    full_spec = pl.BlockSpec((d, d), lambda i: (0, 0))
    b_spec = pl.BlockSpec((1, d), lambda i: (0, 0))
    return pl.pallas_call(
        body,
        grid=grid,
        in_specs=[row_spec, row_spec, row_spec, full_spec, b_spec,
                  row_spec, row_spec, row_spec, row_spec, deg_spec, deg_spec],
        out_specs=[row_spec, row_spec],
        out_shape=[
            jax.ShapeDtypeStruct((n, d), jnp.float32),
            jax.ShapeDtypeStruct((n, d), jnp.float32),
        ],
    )(x, m, root, w, b2, pr0, pr1, pm0, pm1, pd0, pd1)


def kernel(x, m, root, edge_index, W, b, depth):
    n, d = x.shape
    e = edge_index.shape[1]
    assert e % CH == 0 and n % NS == 0

    ept = -(-e // (NW * CH * RING)) * (CH * RING)
    epad = NW * ept
    src2d = jnp.pad(edge_index[0], (0, epad - e)).reshape(NW, ept // CH, CH)
    dst2d = jnp.pad(edge_index[1], (0, epad - e)).reshape(NW, ept // CH, CH)
    msg = jnp.where(depth == 1, x, m)
    npad = NS * CH * NSTRIPE
    assert npad >= n
    zacc = jnp.zeros((CH, d), jnp.float32)
    onesb = jnp.ones((CH, d), jnp.float32)

    out, pr, pm, pd = _sc_edge_kernel(e, npad, root, m, x, msg, src2d, dst2d,
                                      zacc, onesb)
    new_m, new_root = _tc_combine(
        x, m, root, W, b.reshape(1, d),
        pr[:n], pr[npad:npad + n], pm[:n], pm[npad:npad + n],
        pd[:n], pd[npad:npad + n])
    return out, new_m, new_root


# speedup vs baseline: 5.7972x; 1.0466x over previous
import functools

import jax
import jax.numpy as jnp
from jax import lax
from jax.experimental import pallas as pl
from jax.experimental.pallas import tpu as pltpu
from jax.experimental.pallas import tpu_sc as plsc

NC = 2
NS = 16
NW = NC * NS
CH = 128
RING = 16
NSTRIPE = 5


def _sc_edge_kernel(e, npad, root, m, x, msg, src2d, dst2d, zacc, onesb):
    n, d = root.shape
    _, nchunk, ch = src2d.shape
    ngrp = nchunk // RING
    zr = npad // NS
    assert zr // ch == NSTRIPE

    mesh = plsc.VectorSubcoreMesh(
        core_axis_name="c", subcore_axis_name="s", num_cores=NC, num_subcores=NS
    )

    @functools.partial(
        pl.kernel,
        out_type=(
            jax.ShapeDtypeStruct((e, 3 * d), jnp.float32),
            jax.ShapeDtypeStruct((NC * npad, d), jnp.float32),
            jax.ShapeDtypeStruct((NC * npad, d), jnp.float32),
            jax.ShapeDtypeStruct((NC * npad, d), jnp.float32),
        ),
        mesh=mesh,
        scratch_types=(
            pltpu.VMEM_SHARED((npad, d), jnp.float32),
            pltpu.VMEM((RING, ch), jnp.int32),
            pltpu.VMEM((RING, ch), jnp.int32),
            pltpu.VMEM((ch, d), jnp.float32),
            pltpu.VMEM((ch, d), jnp.float32),
            pltpu.SemaphoreType.DMA,
            pltpu.SemaphoreType.DMA,
            pltpu.SemaphoreType.DMA,
            pltpu.SemaphoreType.DMA,
        ),
    )
    def k(root_h, m_h, x_h, msg_h, src_h, dst_h, zacc_h, ones_h,
          out_h, pr_h, pm_h, pd_h,
          acc, irb, dstb, s0, s1, sg0, sg1, sw0, sw1):
        cid = lax.axis_index("c")
        sid = lax.axis_index("s")
        wid = cid * NS + sid
        ebase = wid * nchunk * ch
        slots = (s0, s1)
        gsem = (sg0, sg1)
        wsem = (sw0, sw1)

        def zero_acc_stripe():
            pltpu.sync_copy(zacc_h, s0)

            def zs(b, carry):
                pltpu.sync_copy(s0, acc.at[pl.ds(sid * zr + b * ch, ch)])
                return carry

            lax.fori_loop(0, NSTRIPE, zs, 0)

        def flush_acc_stripe(dst_h_ref, base):
            def fl(b, carry):
                pltpu.sync_copy(acc.at[pl.ds(sid * zr + b * ch, ch)], s0)
                pltpu.sync_copy(s0, dst_h_ref.at[pl.ds(base + b * ch, ch)])
                return carry

            lax.fori_loop(0, NSTRIPE, fl, 0)

        def edge_phase(tab_h, idx_is_dst, colbase, do_add):
            ih = dst_h if idx_is_dst else src_h

            def chunk_simple(k_, jb):
                sl = 0
                pltpu.async_copy(tab_h.at[irb.at[k_]], slots[sl],
                                 gsem[sl]).wait()
                if colbase is not None:
                    row = pl.ds(jb + k_ * ch, ch)
                    pltpu.sync_copy(slots[sl],
                                    out_h.at[row, pl.ds(colbase * d, d)])
                if do_add:
                    didx = dstb.at[k_] if not idx_is_dst else irb.at[k_]
                    pltpu.sync_copy(slots[sl], acc.at[didx], add=True)

            def grp(g, carry):
                nreal = jnp.clip((e - ebase) // ch - g * RING, 0, RING)

                @pl.when(nreal > 0)
                def _():
                    pltpu.sync_copy(ih.at[wid, pl.ds(g * RING, RING)], irb)
                    if do_add and not idx_is_dst:
                        pltpu.sync_copy(
                            dst_h.at[wid, pl.ds(g * RING, RING)], dstb)
                    jb = ebase + g * RING * ch

                    @pl.when(nreal == RING)
                    def _():
                        gd = [None] * RING
                        wd = [None] * RING
                        gd[0] = pltpu.async_copy(
                            tab_h.at[irb.at[0]], slots[0], gsem[0])
                        for k_ in range(RING):
                            sl = k_ % 2
                            if k_ + 1 < RING:
                                if k_ >= 1 and wd[k_ - 1] is not None:
                                    wd[k_ - 1].wait()
                                gd[k_ + 1] = pltpu.async_copy(
                                    tab_h.at[irb.at[k_ + 1]],
                                    slots[(k_ + 1) % 2], gsem[(k_ + 1) % 2])
                            gd[k_].wait()
                            if colbase is not None:
                                row = pl.ds(jb + k_ * ch, ch)
                                wd[k_] = pltpu.async_copy(
                                    slots[sl],
                                    out_h.at[row, pl.ds(colbase * d, d)],
                                    wsem[sl])
                            if do_add:
                                didx = (dstb.at[k_] if not idx_is_dst
                                        else irb.at[k_])
                                pltpu.sync_copy(slots[sl], acc.at[didx],
                                                add=True)
                        for k_ in (RING - 2, RING - 1):
                            if wd[k_] is not None:
                                wd[k_].wait()

                    @pl.when(nreal < RING)
                    def _():
                        def simple(k_, c2):
                            chunk_simple(k_, jb)
                            return c2

                        lax.fori_loop(0, nreal, simple, 0)

                return carry

            lax.fori_loop(0, ngrp, grp, 0)

        zero_acc_stripe()
        plsc.subcore_barrier()

        edge_phase(root_h, False, 0, True)
        plsc.subcore_barrier()
        flush_acc_stripe(pr_h, cid * npad + sid * zr)
        zero_acc_stripe()
        plsc.subcore_barrier()

        edge_phase(m_h, False, 1, False)
        edge_phase(x_h, True, 2, False)
        edge_phase(msg_h, False, None, True)
        plsc.subcore_barrier()
        flush_acc_stripe(pm_h, cid * npad + sid * zr)
        zero_acc_stripe()
        plsc.subcore_barrier()

        pltpu.sync_copy(ones_h, s0)

        def deg_grp(g, carry):
            nreal = jnp.clip((e - ebase) // ch - g * RING, 0, RING)

            @pl.when(nreal > 0)
            def _():
                pltpu.sync_copy(dst_h.at[wid, pl.ds(g * RING, RING)], dstb)

                def f(k_, c2):
                    pltpu.sync_copy(s0, acc.at[dstb.at[k_]], add=True)
                    return c2

                lax.fori_loop(0, nreal, f, 0)

            return carry

        lax.fori_loop(0, ngrp, deg_grp, 0)
        plsc.subcore_barrier()
        flush_acc_stripe(pd_h, cid * npad + sid * zr)

    return k(root, m, x, msg, src2d, dst2d, zacc, onesb)


def _tc_combine(x, m, root, w, b2, pr0, pr1, pm0, pm1, pd0, pd1):
    n, d = x.shape
    bn = 1000
    grid = (n // bn,)

    def body(x_r, m_r, root_r, w_r, b_r, pr0_r, pr1_r, pm0_r, pm1_r,
             pd0_r, pd1_r, newm_r, newroot_r):
        degv = pd0_r[...] + pd1_r[...]
        deg = degv[:, 0:1]
        denom = jnp.maximum(deg, 1.0)
        has = deg > 0.0
        segm = pm0_r[...] + pm1_r[...]
        segr = pr0_r[...] + pr1_r[...]
        xm = x_r[...] + segm / denom
        h = lax.dot_general(xm, w_r[...], (((1,), (1,)), ((), ())),
                            preferred_element_type=jnp.float32)
        h = h + 2.0 * b_r[...]
        newm_r[...] = jnp.where(has, jnp.maximum(h, 0.0), m_r[...])
        newroot_r[...] = jnp.where(has, segr / denom, root_r[...])

    row_spec = pl.BlockSpec((bn, d), lambda i: (i, 0))
    deg_spec = pl.BlockSpec((bn, d), lambda i: (i, 0))
    full_spec = pl.BlockSpec((d, d), lambda i: (0, 0))
    b_spec = pl.BlockSpec((1, d), lambda i: (0, 0))
    return pl.pallas_call(
        body,
        grid=grid,
        in_specs=[row_spec, row_spec, row_spec, full_spec, b_spec,
                  row_spec, row_spec, row_spec, row_spec, deg_spec, deg_spec],
        out_specs=[row_spec, row_spec],
        out_shape=[
            jax.ShapeDtypeStruct((n, d), jnp.float32),
            jax.ShapeDtypeStruct((n, d), jnp.float32),
        ],
    )(x, m, root, w, b2, pr0, pr1, pm0, pm1, pd0, pd1)


def kernel(x, m, root, edge_index, W, b, depth):
    n, d = x.shape
    e = edge_index.shape[1]
    assert e % CH == 0 and n % NS == 0

    ept = -(-e // (NW * CH * RING)) * (CH * RING)
    epad = NW * ept
    src2d = jnp.pad(edge_index[0], (0, epad - e)).reshape(NW, ept // CH, CH)
    dst2d = jnp.pad(edge_index[1], (0, epad - e)).reshape(NW, ept // CH, CH)
    msg = jnp.where(depth == 1, x, m)
    npad = NS * CH * NSTRIPE
    assert npad >= n
    zacc = jnp.zeros((CH, d), jnp.float32)
    onesb = jnp.ones((CH, d), jnp.float32)

    out, pr, pm, pd = _sc_edge_kernel(e, npad, root, m, x, msg, src2d, dst2d,
                                      zacc, onesb)
    new_m, new_root = _tc_combine(
        x, m, root, W, b.reshape(1, d),
        pr[:n], pr[npad:npad + n], pm[:n], pm[npad:npad + n],
        pd[:n], pd[npad:npad + n])
    return out, new_m, new_root


# speedup vs baseline: 5.9715x; 1.0301x over previous
import functools

import jax
import jax.numpy as jnp
from jax import lax
from jax.experimental import pallas as pl
from jax.experimental.pallas import tpu as pltpu
from jax.experimental.pallas import tpu_sc as plsc

NC = 2
NS = 16
NW = NC * NS
CH = 128
RING = 40
NSTRIPE = 5


def _sc_edge_kernel(e, npad, root, m, x, msg, src2d, dst2d, zacc, onesb):
    n, d = root.shape
    _, nchunk, ch = src2d.shape
    ngrp = nchunk // RING
    zr = npad // NS
    assert zr // ch == NSTRIPE

    mesh = plsc.VectorSubcoreMesh(
        core_axis_name="c", subcore_axis_name="s", num_cores=NC, num_subcores=NS
    )

    @functools.partial(
        pl.kernel,
        out_type=(
            jax.ShapeDtypeStruct((e, 3 * d), jnp.float32),
            jax.ShapeDtypeStruct((NC * npad, d), jnp.float32),
            jax.ShapeDtypeStruct((NC * npad, d), jnp.float32),
            jax.ShapeDtypeStruct((NC * npad, d), jnp.float32),
        ),
        mesh=mesh,
        scratch_types=(
            pltpu.VMEM_SHARED((npad, d), jnp.float32),
            pltpu.VMEM((RING, ch), jnp.int32),
            pltpu.VMEM((RING, ch), jnp.int32),
            pltpu.VMEM((ch, d), jnp.float32),
            pltpu.VMEM((ch, d), jnp.float32),
            pltpu.SemaphoreType.DMA,
            pltpu.SemaphoreType.DMA,
            pltpu.SemaphoreType.DMA,
            pltpu.SemaphoreType.DMA,
        ),
    )
    def k(root_h, m_h, x_h, msg_h, src_h, dst_h, zacc_h, ones_h,
          out_h, pr_h, pm_h, pd_h,
          acc, irb, dstb, s0, s1, sg0, sg1, sw0, sw1):
        cid = lax.axis_index("c")
        sid = lax.axis_index("s")
        wid = cid * NS + sid
        ebase = wid * nchunk * ch
        slots = (s0, s1)
        gsem = (sg0, sg1)
        wsem = (sw0, sw1)

        def zero_acc_stripe():
            pltpu.sync_copy(zacc_h, s0)

            def zs(b, carry):
                pltpu.sync_copy(s0, acc.at[pl.ds(sid * zr + b * ch, ch)])
                return carry

            lax.fori_loop(0, NSTRIPE, zs, 0)

        def flush_acc_stripe(dst_h_ref, base):
            def fl(b, carry):
                pltpu.sync_copy(acc.at[pl.ds(sid * zr + b * ch, ch)], s0)
                pltpu.sync_copy(s0, dst_h_ref.at[pl.ds(base + b * ch, ch)])
                return carry

            lax.fori_loop(0, NSTRIPE, fl, 0)

        def edge_phase(tab_h, idx_is_dst, colbase, do_add):
            ih = dst_h if idx_is_dst else src_h

            def chunk_simple(k_, jb):
                sl = 0
                pltpu.async_copy(tab_h.at[irb.at[k_]], slots[sl],
                                 gsem[sl]).wait()
                if colbase is not None:
                    row = pl.ds(jb + k_ * ch, ch)
                    pltpu.sync_copy(slots[sl],
                                    out_h.at[row, pl.ds(colbase * d, d)])
                if do_add:
                    didx = dstb.at[k_] if not idx_is_dst else irb.at[k_]
                    pltpu.sync_copy(slots[sl], acc.at[didx], add=True)

            def grp(g, carry):
                nreal = jnp.clip((e - ebase) // ch - g * RING, 0, RING)

                @pl.when(nreal > 0)
                def _():
                    pltpu.sync_copy(ih.at[wid, pl.ds(g * RING, RING)], irb)
                    if do_add and not idx_is_dst:
                        pltpu.sync_copy(
                            dst_h.at[wid, pl.ds(g * RING, RING)], dstb)
                    jb = ebase + g * RING * ch

                    @pl.when(nreal == RING)
                    def _():
                        gd = [None] * RING
                        wd = [None] * RING
                        gd[0] = pltpu.async_copy(
                            tab_h.at[irb.at[0]], slots[0], gsem[0])
                        for k_ in range(RING):
                            sl = k_ % 2
                            if k_ + 1 < RING:
                                if k_ >= 1 and wd[k_ - 1] is not None:
                                    wd[k_ - 1].wait()
                                gd[k_ + 1] = pltpu.async_copy(
                                    tab_h.at[irb.at[k_ + 1]],
                                    slots[(k_ + 1) % 2], gsem[(k_ + 1) % 2])
                            gd[k_].wait()
                            if colbase is not None:
                                row = pl.ds(jb + k_ * ch, ch)
                                wd[k_] = pltpu.async_copy(
                                    slots[sl],
                                    out_h.at[row, pl.ds(colbase * d, d)],
                                    wsem[sl])
                            if do_add:
                                didx = (dstb.at[k_] if not idx_is_dst
                                        else irb.at[k_])
                                pltpu.sync_copy(slots[sl], acc.at[didx],
                                                add=True)
                        for k_ in (RING - 2, RING - 1):
                            if wd[k_] is not None:
                                wd[k_].wait()

                    @pl.when(nreal < RING)
                    def _():
                        def simple(k_, c2):
                            chunk_simple(k_, jb)
                            return c2

                        lax.fori_loop(0, nreal, simple, 0)

                return carry

            lax.fori_loop(0, ngrp, grp, 0)

        zero_acc_stripe()
        plsc.subcore_barrier()

        edge_phase(root_h, False, 0, True)
        plsc.subcore_barrier()
        flush_acc_stripe(pr_h, cid * npad + sid * zr)
        zero_acc_stripe()
        plsc.subcore_barrier()

        edge_phase(m_h, False, 1, False)
        edge_phase(x_h, True, 2, False)
        edge_phase(msg_h, False, None, True)
        plsc.subcore_barrier()
        flush_acc_stripe(pm_h, cid * npad + sid * zr)
        zero_acc_stripe()
        plsc.subcore_barrier()

        pltpu.sync_copy(ones_h, s0)

        def deg_grp(g, carry):
            nreal = jnp.clip((e - ebase) // ch - g * RING, 0, RING)

            @pl.when(nreal > 0)
            def _():
                pltpu.sync_copy(dst_h.at[wid, pl.ds(g * RING, RING)], dstb)

                def f(k_, c2):
                    pltpu.sync_copy(s0, acc.at[dstb.at[k_]], add=True)
                    return c2

                lax.fori_loop(0, nreal, f, 0)

            return carry

        lax.fori_loop(0, ngrp, deg_grp, 0)
        plsc.subcore_barrier()
        flush_acc_stripe(pd_h, cid * npad + sid * zr)

    return k(root, m, x, msg, src2d, dst2d, zacc, onesb)


def _tc_combine(x, m, root, w, b2, pr0, pr1, pm0, pm1, pd0, pd1):
    n, d = x.shape
    bn = 1000
    grid = (n // bn,)

    def body(x_r, m_r, root_r, w_r, b_r, pr0_r, pr1_r, pm0_r, pm1_r,
             pd0_r, pd1_r, newm_r, newroot_r):
        degv = pd0_r[...] + pd1_r[...]
        deg = degv[:, 0:1]
        denom = jnp.maximum(deg, 1.0)
        has = deg > 0.0
        segm = pm0_r[...] + pm1_r[...]
        segr = pr0_r[...] + pr1_r[...]
        xm = x_r[...] + segm / denom
        h = lax.dot_general(xm, w_r[...], (((1,), (1,)), ((), ())),
                            preferred_element_type=jnp.float32)
        h = h + 2.0 * b_r[...]
        newm_r[...] = jnp.where(has, jnp.maximum(h, 0.0), m_r[...])
        newroot_r[...] = jnp.where(has, segr / denom, root_r[...])

    row_spec = pl.BlockSpec((bn, d), lambda i: (i, 0))
    deg_spec = pl.BlockSpec((bn, d), lambda i: (i, 0))
    full_spec = pl.BlockSpec((d, d), lambda i: (0, 0))
    b_spec = pl.BlockSpec((1, d), lambda i: (0, 0))
    return pl.pallas_call(
        body,
        grid=grid,
        in_specs=[row_spec, row_spec, row_spec, full_spec, b_spec,
                  row_spec, row_spec, row_spec, row_spec, deg_spec, deg_spec],
        out_specs=[row_spec, row_spec],
        out_shape=[
            jax.ShapeDtypeStruct((n, d), jnp.float32),
            jax.ShapeDtypeStruct((n, d), jnp.float32),
        ],
    )(x, m, root, w, b2, pr0, pr1, pm0, pm1, pd0, pd1)


def kernel(x, m, root, edge_index, W, b, depth):
    n, d = x.shape
    e = edge_index.shape[1]
    assert e % CH == 0 and n % NS == 0

    ept = -(-e // (NW * CH * RING)) * (CH * RING)
    epad = NW * ept
    src2d = jnp.pad(edge_index[0], (0, epad - e)).reshape(NW, ept // CH, CH)
    dst2d = jnp.pad(edge_index[1], (0, epad - e)).reshape(NW, ept // CH, CH)
    msg = jnp.where(depth == 1, x, m)
    npad = NS * CH * NSTRIPE
    assert npad >= n
    zacc = jnp.zeros((CH, d), jnp.float32)
    onesb = jnp.ones((CH, d), jnp.float32)

    out, pr, pm, pd = _sc_edge_kernel(e, npad, root, m, x, msg, src2d, dst2d,
                                      zacc, onesb)
    new_m, new_root = _tc_combine(
        x, m, root, W, b.reshape(1, d),
        pr[:n], pr[npad:npad + n], pm[:n], pm[npad:npad + n],
        pd[:n], pd[npad:npad + n])
    return out, new_m, new_root
